# baseline (speedup 1.0000x reference)
import functools

import jax
import jax.numpy as jnp
from jax import lax
from jax.experimental import pallas as pl
from jax.experimental.pallas import tpu as pltpu
from jax.experimental.pallas import tpu_sc as plsc

LANES = 16
NBUF = 2


def _build_sc_embed(n, emb, chunk, n_chunks, rows_per_w, nc):
    mesh = plsc.VectorSubcoreMesh(core_axis_name="c", subcore_axis_name="s")
    n_groups = n_chunks // NBUF

    scratch = [
        pltpu.VMEM((rows_per_w,), jnp.int32),
        pltpu.VMEM((rows_per_w,), jnp.int32),
    ]
    for _ in range(NBUF):
        scratch += [
            pltpu.VMEM((chunk, emb), jnp.float32),
            pltpu.VMEM((chunk, emb), jnp.float32),
        ]
    scratch += [pltpu.SemaphoreType.DMA] * (4 * NBUF)

    @functools.partial(
        pl.kernel,
        out_type=(
            jax.ShapeDtypeStruct((n, emb), jnp.float32),
            jax.ShapeDtypeStruct((n, emb), jnp.float32),
        ),
        mesh=mesh,
        scratch_types=scratch,
    )
    def body(tok_hbm, pos_hbm, emb_hbm, cache_hbm, x_hbm, pe_hbm, *s):
        tok_idx, pos_idx = s[0], s[1]
        bufs = [s[2 + 2 * b:2 + 2 * b + 2] for b in range(NBUF)]
        sems = [s[2 + 2 * NBUF + 4 * b:2 + 2 * NBUF + 4 * b + 4]
                for b in range(NBUF)]

        wid = lax.axis_index("s") * nc + lax.axis_index("c")
        wbase = wid * rows_per_w

        pltpu.sync_copy(tok_hbm.at[pl.ds(wbase, rows_per_w)], tok_idx)
        pltpu.sync_copy(pos_hbm.at[pl.ds(wbase, rows_per_w)], pos_idx)

        def issue(c, b):
            tb, pb = bufs[b]
            sem_gt, sem_gp, _, _ = sems[b]
            off = c * chunk
            pltpu.async_copy(emb_hbm.at[tok_idx.at[pl.ds(off, chunk)]],
                             tb, sem_gt)
            pltpu.async_copy(cache_hbm.at[pos_idx.at[pl.ds(off, chunk)]],
                             pb, sem_gp)

        def add_rows(tb, pb):
            def row_body(r, carry):
                for i in range(emb // LANES):
                    sl = pl.ds(i * LANES, LANES)
                    plsc.addupdate(tb.at[r, sl], pb[r, sl])
                return carry
            lax.fori_loop(0, chunk, row_body, 0)

        for b in range(NBUF):
            issue(b, b)

        def group_body(g, carry):
            for b in range(NBUF):
                c = g * NBUF + b
                tb, pb = bufs[b]
                sem_gt, sem_gp, sem_wx, sem_wp = sems[b]
                base = wbase + c * chunk
                pltpu.make_async_copy(
                    cache_hbm.at[pos_idx.at[pl.ds(0, chunk)]], pb,
                    sem_gp).wait()
                pltpu.async_copy(pb, pe_hbm.at[pl.ds(base, chunk)], sem_wp)
                pltpu.make_async_copy(
                    emb_hbm.at[tok_idx.at[pl.ds(0, chunk)]], tb,
                    sem_gt).wait()
                add_rows(tb, pb)
                pltpu.async_copy(tb, x_hbm.at[pl.ds(base, chunk)], sem_wx)

                nxt = c + NBUF

                @pl.when(nxt < n_chunks)
                def _prefetch():
                    pltpu.make_async_copy(
                        tb, x_hbm.at[pl.ds(0, chunk)], sem_wx).wait()
                    pltpu.make_async_copy(
                        pb, pe_hbm.at[pl.ds(0, chunk)], sem_wp).wait()
                    issue(nxt, b)
            return carry

        lax.fori_loop(0, n_groups, group_body, 0)

        for b in range(NBUF):
            tb, pb = bufs[b]
            _, _, sem_wx, sem_wp = sems[b]
            pltpu.make_async_copy(tb, x_hbm.at[pl.ds(0, chunk)], sem_wx).wait()
            pltpu.make_async_copy(pb, pe_hbm.at[pl.ds(0, chunk)], sem_wp).wait()

    return body


def kernel(tokens, mask, pos_ids, embedding, pos_emb_cache):
    bs, seq = tokens.shape
    _, emb = embedding.shape
    n = bs * seq

    info = plsc.get_sparse_core_info()
    nc, ns = info.num_cores, info.num_subcores
    nw = nc * ns
    rows_per_w = n // nw
    chunk = 16
    n_chunks = rows_per_w // chunk

    tok_flat = tokens.reshape(n).astype(jnp.int32)
    pos_flat = pos_ids.reshape(n).astype(jnp.int32)

    body = _build_sc_embed(n, emb, chunk, n_chunks, rows_per_w, nc)
    x_flat, pe_flat = body(tok_flat, pos_flat, embedding, pos_emb_cache)
    x = x_flat.reshape(bs, seq, emb)
    pe = pe_flat.reshape(bs, seq, emb)
    return (x, mask, pe)


# speedup vs baseline: 1.3543x; 1.3543x over previous
import functools

import jax
import jax.numpy as jnp
from jax import lax
from jax.experimental import pallas as pl
from jax.experimental.pallas import tpu as pltpu
from jax.experimental.pallas import tpu_sc as plsc

LANES = 16
NSLOT = 3


def _build_sc_embed(n, emb, chunk, n_chunks, rows_per_w, nc):
    mesh = plsc.VectorSubcoreMesh(core_axis_name="c", subcore_axis_name="s")

    scratch = []
    for _ in range(NSLOT):
        scratch += [
            pltpu.VMEM((chunk,), jnp.int32),
            pltpu.VMEM((chunk,), jnp.int32),
            pltpu.VMEM((chunk, emb), jnp.float32),
            pltpu.VMEM((chunk, emb), jnp.float32),
        ]
    scratch += [pltpu.SemaphoreType.DMA] * (6 * NSLOT)

    @functools.partial(
        pl.kernel,
        out_type=(
            jax.ShapeDtypeStruct((n, emb), jnp.float32),
            jax.ShapeDtypeStruct((n, emb), jnp.float32),
        ),
        mesh=mesh,
        scratch_types=scratch,
    )
    def body(tok_hbm, pos_hbm, emb_hbm, cache_hbm, x_hbm, pe_hbm, *s):
        bufs = [s[4 * j:4 * j + 4] for j in range(NSLOT)]
        sems = [s[4 * NSLOT + 6 * j:4 * NSLOT + 6 * j + 6]
                for j in range(NSLOT)]

        wid = lax.axis_index("s") * nc + lax.axis_index("c")
        wbase = wid * rows_per_w

        def copy_idx(c, j):
            ti, pi, _, _ = bufs[j]
            base = wbase + c * chunk
            pltpu.async_copy(tok_hbm.at[pl.ds(base, chunk)], ti, sems[j][0])
            pltpu.async_copy(pos_hbm.at[pl.ds(base, chunk)], pi, sems[j][1])

        def issue_gather(j):
            ti, pi, tb, pb = bufs[j]
            pltpu.make_async_copy(
                tok_hbm.at[pl.ds(0, chunk)], ti, sems[j][0]).wait()
            pltpu.make_async_copy(
                pos_hbm.at[pl.ds(0, chunk)], pi, sems[j][1]).wait()
            pltpu.async_copy(emb_hbm.at[ti], tb, sems[j][2])
            pltpu.async_copy(cache_hbm.at[pi], pb, sems[j][3])

        def wait_writes(j):
            _, _, tb, pb = bufs[j]
            pltpu.make_async_copy(tb, x_hbm.at[pl.ds(0, chunk)],
                                  sems[j][4]).wait()
            pltpu.make_async_copy(pb, pe_hbm.at[pl.ds(0, chunk)],
                                  sems[j][5]).wait()

        def add_rows(tb, pb):
            def row_body(r, carry):
                for i in range(emb // LANES):
                    sl = pl.ds(i * LANES, LANES)
                    plsc.addupdate(tb.at[r, sl], pb[r, sl])
                return carry
            lax.fori_loop(0, chunk, row_body, 0)

        def process(c, j):
            ti, pi, tb, pb = bufs[j]
            base = wbase + c * chunk
            pltpu.make_async_copy(
                cache_hbm.at[pi], pb, sems[j][3]).wait()
            pltpu.async_copy(pb, pe_hbm.at[pl.ds(base, chunk)], sems[j][5])
            pltpu.make_async_copy(
                emb_hbm.at[ti], tb, sems[j][2]).wait()
            add_rows(tb, pb)
            pltpu.async_copy(tb, x_hbm.at[pl.ds(base, chunk)], sems[j][4])

        for c in range(NSLOT):
            copy_idx(c, c)
        for c in range(2):
            issue_gather(c)

        n_loop = n_chunks - 2

        def group_body(g, carry):
            for j in range(NSLOT):
                c = g * NSLOT + j
                process(c, j)

                @pl.when(c + NSLOT < n_chunks)
                def _idx_prefetch():
                    copy_idx(c + NSLOT, j)

                k = (j + 2) % NSLOT

                @pl.when(c >= 1)
                def _wait_prev_writes():
                    wait_writes(k)

                issue_gather(k)
            return carry

        lax.fori_loop(0, n_loop // NSLOT, group_body, 0)

        for c in range(n_chunks - 2, n_chunks):
            process(c, c % NSLOT)

        for j in range(NSLOT):
            wait_writes(j)

    return body


def kernel(tokens, mask, pos_ids, embedding, pos_emb_cache):
    bs, seq = tokens.shape
    _, emb = embedding.shape
    n = bs * seq

    info = plsc.get_sparse_core_info()
    nc, ns = info.num_cores, info.num_subcores
    nw = nc * ns
    rows_per_w = n // nw
    chunk = 16
    n_chunks = rows_per_w // chunk

    tok_flat = tokens.reshape(n).astype(jnp.int32)
    pos_flat = pos_ids.reshape(n).astype(jnp.int32)

    body = _build_sc_embed(n, emb, chunk, n_chunks, rows_per_w, nc)
    x_flat, pe_flat = body(tok_flat, pos_flat, embedding, pos_emb_cache)
    x = x_flat.reshape(bs, seq, emb)
    pe = pe_flat.reshape(bs, seq, emb)
    return (x, mask, pe)
